# Initial kernel scaffold; baseline (speedup 1.0000x reference)
import jax
import jax.numpy as jnp
from jax.experimental import pallas as pl


def kernel(feats, edge_index, W1_0, Wself_0, b_0, W1_1, Wself_1, b_1, Wc1, bc1, Wc2, bc2, Wc3, bc3):
    raise NotImplementedError("write your pallas kernel here")


# speedup vs baseline: 9.5036x; 9.5036x over previous
import functools

import jax
import jax.numpy as jnp
from jax import lax
from jax.experimental import pallas as pl
from jax.experimental.pallas import tpu as pltpu
from jax.experimental.pallas import tpu_sc as plsc

N = 100000
E = 3200000
H = 128
D = 16
G = 8
EB = 1024
SUB = EB // 128
NB = E // EB
NC = 2
NS = 16
NW = NC * NS
ZCH = 1000
NZC = N // ZCH
BLK = 2000
NBLK = N // BLK

_MESH = plsc.VectorSubcoreMesh(core_axis_name="c", subcore_axis_name="s")


def _node_chunks(s, body):
    def step(i, carry):
        idx = s + NS * i
        @pl.when(idx < NZC)
        def _():
            body(idx * ZCH)
        return carry
    lax.fori_loop(0, (NZC + NS - 1) // NS, step, 0, unroll=False)


@functools.partial(
    pl.kernel,
    out_type=jax.ShapeDtypeStruct((NC, N, D), jnp.float32),
    mesh=_MESH,
    scratch_types=[
        pltpu.VMEM_SHARED((N, D), jnp.float32),
        pltpu.VMEM((SUB, 128), jnp.int32),
        pltpu.VMEM((SUB, 128), jnp.int32),
        pltpu.VMEM((SUB, 128, D), jnp.float32),
        pltpu.SemaphoreType.DMA,
    ],
    compiler_params=pltpu.CompilerParams(use_tc_tiling_on_sc=False),
)
def _edge_acc(feats16, src2, dst2, zeros, acc_out, acc, sidx, ddst, rows, sem):
    c = lax.axis_index("c")
    s = lax.axis_index("s")
    wid = c * NS + s

    _node_chunks(s, lambda base: pltpu.sync_copy(
        zeros.at[pl.ds(base, ZCH)], acc.at[pl.ds(base, ZCH)]))
    plsc.subcore_barrier()

    def block(i, carry):
        b = wid + NW * i
        @pl.when(b < NB)
        def _():
            pltpu.sync_copy(src2.at[pl.ds(b * SUB, SUB)], sidx)
            pltpu.sync_copy(dst2.at[pl.ds(b * SUB, SUB)], ddst)
            descs = [pltpu.async_copy(feats16.at[sidx.at[j]], rows.at[j], sem)
                     for j in range(SUB)]
            for dsc in descs:
                dsc.wait()
            for j in range(SUB):
                pltpu.sync_copy(rows.at[j], acc.at[ddst.at[j]], add=True)
        return carry
    lax.fori_loop(0, (NB + NW - 1) // NW, block, 0, unroll=False)
    plsc.subcore_barrier()

    _node_chunks(s, lambda base: pltpu.sync_copy(
        acc.at[pl.ds(base, ZCH)], acc_out.at[c, pl.ds(base, ZCH), :]))


@functools.partial(
    pl.kernel,
    out_type=jax.ShapeDtypeStruct((G, N, D), jnp.float32),
    mesh=_MESH,
    scratch_types=[
        pltpu.VMEM_SHARED((N, D), jnp.float32),
        pltpu.VMEM((SUB, 128), jnp.int32),
        pltpu.VMEM((SUB, 128), jnp.int32),
        pltpu.VMEM((SUB, 128), jnp.int32),
        pltpu.VMEM((SUB, 128, D), jnp.float32),
        pltpu.SemaphoreType.DMA,
    ],
    compiler_params=pltpu.CompilerParams(use_tc_tiling_on_sc=False),
)
def _seg_sum(h1v, src2, dst2, zeros, s_out, acc, sidx, gidx, ddst, rows, sem):
    c = lax.axis_index("c")
    s = lax.axis_index("s")

    for k in range(G // NC):
        g = c * (G // NC) + k

        _node_chunks(s, lambda base: pltpu.sync_copy(
            zeros.at[pl.ds(base, ZCH)], acc.at[pl.ds(base, ZCH)]))
        plsc.subcore_barrier()

        def block(i, carry):
            b = s + NS * i
            @pl.when(b < NB)
            def _():
                pltpu.sync_copy(src2.at[pl.ds(b * SUB, SUB)], sidx)
                pltpu.sync_copy(dst2.at[pl.ds(b * SUB, SUB)], ddst)
                for j in range(SUB):
                    for q in range(8):
                        sl = (j, pl.ds(q * 16, 16))
                        gidx[sl] = sidx[sl] * G + g
                descs = [pltpu.async_copy(h1v.at[gidx.at[j]], rows.at[j], sem)
                         for j in range(SUB)]
                for dsc in descs:
                    dsc.wait()
                for j in range(SUB):
                    pltpu.sync_copy(rows.at[j], acc.at[ddst.at[j]], add=True)
            return carry
        lax.fori_loop(0, (NB + NS - 1) // NS, block, 0, unroll=False)
        plsc.subcore_barrier()

        _node_chunks(s, lambda base: pltpu.sync_copy(
            acc.at[pl.ds(base, ZCH)], s_out.at[g, pl.ds(base, ZCH), :]))
        plsc.subcore_barrier()


def _rb(x):
    return x.astype(jnp.bfloat16).astype(jnp.float32)


def _dense0_body(accA, feats, W1, Ws, b0, h1_ref, dinv_ref):
    a = accA[0] + accA[1]
    deg = jnp.maximum(a[:, 2:3], 1.0)
    dinv = 1.0 / deg
    agg = jnp.dot(a[:, 0:2], W1[...], preferred_element_type=jnp.float32,
                  precision=lax.Precision.HIGHEST) * dinv
    hs = jnp.dot(feats[...], Ws[...], preferred_element_type=jnp.float32,
                 precision=lax.Precision.HIGHEST)
    h1_ref[...] = _rb(jnp.tanh(jnp.tanh(agg + hs + b0[...])))
    dinv_ref[...] = jnp.broadcast_to(dinv, (BLK, D))


def _dense1_body(S1, h1, dinv, W1g, Ws, b1, Wc1, bc1, Wc2, bc2, Wc3, bc3,
                 out_ref, gmax):
    i = pl.program_id(0)
    Sb = S1[...]
    agg = jnp.zeros((BLK, H), jnp.float32)
    for g in range(G):
        agg = agg + jnp.dot(Sb[g], W1g[...][g],
                            preferred_element_type=jnp.float32,
                            precision=lax.Precision.HIGHEST)
    agg = agg * dinv[...][:, 0:1]
    h2 = jnp.tanh(jnp.tanh(agg + jnp.dot(h1[...], Ws[...],
                                         preferred_element_type=jnp.float32,
                                         precision=lax.Precision.HIGHEST)
                           + b1[...]))
    m = jnp.max(h2, axis=0, keepdims=True)

    @pl.when(i == 0)
    def _():
        gmax[...] = m

    @pl.when(i > 0)
    def _():
        gmax[...] = jnp.maximum(gmax[...], m)

    def elu(x):
        return jnp.where(x > 0, x, jnp.exp(x) - 1.0)

    @pl.when(i == pl.num_programs(0) - 1)
    def _():
        gv = _rb(gmax[...])
        z1 = elu(jnp.dot(gv, Wc1[...], preferred_element_type=jnp.float32,
                         precision=lax.Precision.HIGHEST) + bc1[...])
        z2 = elu(jnp.dot(_rb(z1), Wc2[...], preferred_element_type=jnp.float32,
                         precision=lax.Precision.HIGHEST) + bc2[...])
        out_ref[...] = jnp.dot(_rb(z2), Wc3[...], preferred_element_type=jnp.float32,
                               precision=lax.Precision.HIGHEST) + bc3[...]


def _full(shape):
    nd = len(shape)
    return pl.BlockSpec(shape, lambda i, _nd=nd: (0,) * _nd)


def kernel(feats, edge_index, W1_0, Wself_0, b_0, W1_1, Wself_1, b_1,
           Wc1, bc1, Wc2, bc2, Wc3, bc3):
    src = edge_index[0].astype(jnp.int32)
    dst = edge_index[1].astype(jnp.int32)
    src2 = src.reshape(NB * SUB, 128)
    dst2 = dst.reshape(NB * SUB, 128)
    featsb = feats.astype(jnp.bfloat16).astype(jnp.float32)
    feats16 = jnp.concatenate(
        [featsb, jnp.ones((N, 1), jnp.float32), jnp.zeros((N, D - 3), jnp.float32)],
        axis=1)
    zeros = jnp.zeros((N, D), jnp.float32)

    accA = _edge_acc(feats16, src2, dst2, zeros)

    h1, dinv = pl.pallas_call(
        _dense0_body,
        grid=(NBLK,),
        in_specs=[
            pl.BlockSpec((NC, BLK, D), lambda i: (0, i, 0)),
            pl.BlockSpec((BLK, 2), lambda i: (i, 0)),
            _full((2, H)),
            _full((2, H)),
            _full((1, H)),
        ],
        out_specs=[
            pl.BlockSpec((BLK, H), lambda i: (i, 0)),
            pl.BlockSpec((BLK, D), lambda i: (i, 0)),
        ],
        out_shape=[
            jax.ShapeDtypeStruct((N, H), jnp.float32),
            jax.ShapeDtypeStruct((N, D), jnp.float32),
        ],
    )(accA, featsb,
      W1_0.astype(jnp.bfloat16).astype(jnp.float32),
      Wself_0.astype(jnp.bfloat16).astype(jnp.float32),
      b_0.reshape(1, H))

    S1 = _seg_sum(h1.reshape(N * G, D), src2, dst2, zeros)

    out = pl.pallas_call(
        _dense1_body,
        grid=(NBLK,),
        in_specs=[
            pl.BlockSpec((G, BLK, D), lambda i: (0, i, 0)),
            pl.BlockSpec((BLK, H), lambda i: (i, 0)),
            pl.BlockSpec((BLK, D), lambda i: (i, 0)),
            _full((G, D, H)),
            _full((H, H)),
            _full((1, H)),
            _full((H, H)),
            _full((1, H)),
            _full((H, 32)),
            _full((1, 32)),
            _full((32, 1)),
            _full((1, 1)),
        ],
        out_specs=pl.BlockSpec((1, 1), lambda i: (0, 0)),
        out_shape=jax.ShapeDtypeStruct((1, 1), jnp.float32),
        scratch_shapes=[pltpu.VMEM((1, H), jnp.float32)],
        compiler_params=pltpu.CompilerParams(
            dimension_semantics=("arbitrary",)),
    )(S1, h1, dinv,
      W1_1.astype(jnp.bfloat16).astype(jnp.float32).reshape(G, D, H),
      Wself_1.astype(jnp.bfloat16).astype(jnp.float32),
      b_1.reshape(1, H),
      Wc1.astype(jnp.bfloat16).astype(jnp.float32), bc1.reshape(1, H),
      Wc2.astype(jnp.bfloat16).astype(jnp.float32), bc2.reshape(1, 32),
      Wc3.astype(jnp.bfloat16).astype(jnp.float32), bc3.reshape(1, 1))

    return out
